# Initial kernel scaffold; baseline (speedup 1.0000x reference)
import jax
import jax.numpy as jnp
from jax.experimental import pallas as pl


def kernel(x, h1, c1, del_t, edge_index, W_iou, U_iou, b_iou, U_f, W_q, b_q, W_k, b_k, W_c, b_c, a, b, haw_1, haw_2):
    raise NotImplementedError("write your pallas kernel here")


# speedup vs baseline: 6.6257x; 6.6257x over previous
import functools

import jax
import jax.numpy as jnp
import numpy as np
from jax import lax
from jax.experimental import pallas as pl
from jax.experimental.pallas import tpu as pltpu
from jax.experimental.pallas import tpu_sc as plsc

N = 10000
DEG = 16
XS = 128
HS = 128
CURV = 1.0
SQC = float(np.sqrt(CURV))
EPS = 1e-15
INV_SQRT_HS = float(1.0 / np.sqrt(HS))

NC = 2
NS = 16
NW = NC * NS
LANES = 16

PERW = 320
NPAD = NW * PERW
GD = 8
NGROUP = PERW // GD

TW = 512
XW = 256
OW = 384

S_XN, S_T, S_G, S_M, S_A, S_Q, S_KH2, S_U2, S_V2, S_RHO, S_HAW1 = range(11)

_INTERPRET = False
_PREC = lax.Precision.HIGHEST


def _artanh(x):
    return 0.5 * jnp.log((1.0 + jnp.clip(x, -1.0 + 1e-5, 1.0 - 1e-5)) /
                         (1.0 - jnp.clip(x, -1.0 + 1e-5, 1.0 - 1e-5)))


def _norm(x):
    return jnp.clip(jnp.sqrt(jnp.sum(x * x, -1, keepdims=True)), EPS)


def _mobius_add(x, y):
    x2 = jnp.sum(x * x, -1, keepdims=True)
    y2 = jnp.sum(y * y, -1, keepdims=True)
    xy = jnp.sum(x * y, -1, keepdims=True)
    num = (1.0 + 2.0 * CURV * xy + CURV * y2) * x + (1.0 - CURV * x2) * y
    den = 1.0 + 2.0 * CURV * xy + CURV * CURV * x2 * y2
    return num / jnp.clip(den, EPS)


def _expmap0(u):
    n = _norm(u)
    return jnp.tanh(SQC * n) * u / (SQC * n)


def _logmap0(x):
    n = _norm(x)
    return _artanh(SQC * n) * x / (SQC * n)


def _mobius_matvec(M, x):
    mx = lax.dot_general(x, M, (((1,), (1,)), ((), ())), precision=_PREC)
    xn = _norm(x)
    mxn = _norm(mx)
    return jnp.tanh(mxn / xn * _artanh(SQC * xn)) * mx / (mxn * SQC)


def _mobius_pw(w, x):
    wx = w * x
    xn = _norm(x)
    wxn = _norm(wx)
    return jnp.tanh(wxn / xn * _artanh(SQC * xn)) * wx / (wxn * SQC)


def _sigmoid(x):
    return 1.0 / (1.0 + jnp.exp(-x))


def _stage1_body(x_ref, h1_ref, c1_ref, dt_ref,
                 W_q_ref, b_q_ref, W_k_ref, b_k_ref, W_c_ref, b_c_ref,
                 U_f_ref, W_iou_ref, a_ref, b_ref, h1s_ref, h2s_ref,
                 T_ref, Xt_ref, WXI_ref):
    x = x_ref[...]
    h1 = h1_ref[...]
    c1 = c1_ref[...]
    dt = dt_ref[...]
    a = a_ref[0, 0]
    b = b_ref[0, 0]
    haw_1 = h1s_ref[0, 0]
    haw_2 = h2s_ref[0, 0]

    ex = _expmap0(x)
    x_ = _mobius_add(_mobius_matvec(W_q_ref[...], ex), b_q_ref[...])
    x2 = jnp.sum(x_ * x_, -1, keepdims=True)
    kh = _mobius_add(_mobius_matvec(W_k_ref[...], h1), b_k_ref[...])
    kh2 = jnp.sum(kh * kh, -1, keepdims=True)

    xn = jnp.clip(jnp.sqrt(jnp.sum(h1 * h1, -1, keepdims=True)), EPS)
    t = _artanh(SQC * xn)
    u = h1 / (xn * SQC)
    v = jnp.maximum(u, 0.0)
    u2 = jnp.sum(u * u, -1, keepdims=True)
    v2 = jnp.sum(v * v, -1, keepdims=True)
    rho = jnp.sqrt(v2)
    m = jnp.exp(-haw_2 * dt / 60.0)
    A = _artanh(SQC * m)
    g = b * jnp.exp(-a * dt)

    cc = _mobius_add(_mobius_matvec(W_c_ref[...], c1), b_c_ref[...])
    c_sk = _expmap0(jnp.tanh(_logmap0(cc)))
    c_sk_hat = _mobius_pw(c_sk, g)
    c_Tk = _mobius_add(-c_sk, c1)
    c_kt = _mobius_add(c_Tk, c_sk_hat)
    f = _sigmoid(_logmap0(_mobius_matvec(U_f_ref[...], h1)))
    v_c = _mobius_pw(f, c_kt)
    lam_c = 2.0 / jnp.clip(1.0 - CURV * jnp.sum(v_c * v_c, -1, keepdims=True), EPS)
    p = lam_c * v_c
    q = lam_c - 1.0

    bn = x.shape[0]
    haw1_col = jnp.full((bn, 1), haw_1, jnp.float32)
    zpad = jnp.zeros((bn, TW - 3 * HS - 11), jnp.float32)
    scal = jnp.concatenate(
        [xn, t, g, m, A, q, kh2, u2, v2, rho, haw1_col, zpad], axis=1)
    T_ref[...] = jnp.concatenate([kh, u, p, scal], axis=1)
    Xt_ref[...] = jnp.concatenate(
        [x_, x2, jnp.zeros((bn, XW - HS - 1), jnp.float32)], axis=1)
    WXI_ref[...] = _mobius_matvec(W_iou_ref[...], ex)


def _stage1(x, h1, c1, dt, W_q, b_q, W_k, b_k, W_c, b_c, U_f, W_iou,
            a, b, haw_1, haw_2):
    BN = 1000
    grid = (N // BN,)
    row = lambda i: (i, 0)
    whole = lambda i: (0, 0)
    return pl.pallas_call(
        _stage1_body,
        grid=grid,
        in_specs=[
            pl.BlockSpec((BN, XS), row),
            pl.BlockSpec((BN, HS), row),
            pl.BlockSpec((BN, HS), row),
            pl.BlockSpec((BN, 1), row),
            pl.BlockSpec((HS, XS), whole),
            pl.BlockSpec((1, HS), whole),
            pl.BlockSpec((HS, HS), whole),
            pl.BlockSpec((1, HS), whole),
            pl.BlockSpec((HS, HS), whole),
            pl.BlockSpec((1, HS), whole),
            pl.BlockSpec((HS, HS), whole),
            pl.BlockSpec((3 * HS, XS), whole),
            pl.BlockSpec((1, 1), whole),
            pl.BlockSpec((1, 1), whole),
            pl.BlockSpec((1, 1), whole),
            pl.BlockSpec((1, 1), whole),
        ],
        out_specs=[
            pl.BlockSpec((BN, TW), row),
            pl.BlockSpec((BN, XW), row),
            pl.BlockSpec((BN, 3 * HS), row),
        ],
        out_shape=[
            jax.ShapeDtypeStruct((N, TW), jnp.float32),
            jax.ShapeDtypeStruct((N, XW), jnp.float32),
            jax.ShapeDtypeStruct((N, 3 * HS), jnp.float32),
        ],
        interpret=_INTERPRET,
    )(x, h1, c1, dt, W_q, b_q, W_k, b_k, W_c, b_c, U_f, W_iou,
      a, b, haw_1, haw_2)


def _ln16(x):
    bits = plsc.bitcast(x, jnp.int32)
    e = ((bits >> 23) & 0xFF) - 127
    m = plsc.bitcast((bits & 0x007FFFFF) | 0x3F800000, jnp.float32)
    big = m > 1.4142135623730951
    m = jnp.where(big, m * 0.5, m)
    e = e + big.astype(jnp.int32)
    s = (m - 1.0) / (m + 1.0)
    s2 = s * s
    poly = 1.0 + s2 * (0.3333333333 + s2 * (0.2 + s2 * (0.14285714 + s2 * 0.11111111)))
    return 2.0 * s * poly + e.astype(jnp.float32) * 0.6931471805599453


def _sqrt16(x):
    xc = jnp.maximum(x, 1e-35)
    y = plsc.bitcast(jnp.int32(0x5F3759DF) - (plsc.bitcast(xc, jnp.int32) >> 1),
                     jnp.float32)
    for _ in range(3):
        y = y * (1.5 - 0.5 * xc * y * y)
    return x * y


def _tanh16(z):
    ez = jnp.exp(-2.0 * z)
    return (1.0 - ez) / (1.0 + ez)


def _sc_edge_body(T_hbm, Xt_hbm, SRC_hbm, O_hbm,
                  idx_v, rows_v, xt_v, out_v, sem):
    wid = lax.axis_index("s") * NC + lax.axis_index("c")
    base = wid * PERW
    lane = lax.iota(jnp.int32, LANES)

    def group(gi, _):
        d0 = base + gi * GD
        pltpu.sync_copy(SRC_hbm.at[pl.ds(d0 * DEG, GD * DEG)], idx_v)
        pltpu.async_copy(T_hbm.at[idx_v], rows_v, sem).wait()
        pltpu.sync_copy(Xt_hbm.at[pl.ds(d0, GD)], xt_v)

        def dst_body(k, _):
            xch = [xt_v[k, pl.ds(c * LANES, LANES)] for c in range(8)]
            dots = jnp.zeros((LANES,), jnp.float32)
            for e in range(DEG):
                r = k * DEG + e
                acc = rows_v[r, pl.ds(0, LANES)] * xch[0]
                for c in range(1, 8):
                    acc = acc + rows_v[r, pl.ds(c * LANES, LANES)] * xch[c]
                dots = jnp.where(lane == e, jnp.sum(acc), dots)

            ridx = lane + (k * DEG)

            def gcol(slot):
                col = jnp.full((LANES,), 3 * HS + slot, jnp.int32)
                return plsc.load_gather(rows_v, [ridx, col])

            xn_s = gcol(S_XN)
            t_s = gcol(S_T)
            g_s = gcol(S_G)
            m_s = gcol(S_M)
            A_s = gcol(S_A)
            q_s = gcol(S_Q)
            kh2_s = gcol(S_KH2)
            u2_s = gcol(S_U2)
            v2_s = gcol(S_V2)
            rho_s = gcol(S_RHO)
            haw1 = gcol(S_HAW1)

            x2d = xt_v[k, pl.ds(HS, LANES)][0]
            P = 1.0 - 2.0 * CURV * dots + CURV * kh2_s
            Q = 1.0 - CURV * x2d
            den = jnp.maximum(1.0 - 2.0 * CURV * dots + CURV * CURV * x2d * kh2_s, EPS)
            nrm2 = P * P * x2d + Q * Q * kh2_s - 2.0 * P * Q * dots
            rr = _sqrt16(jnp.maximum(nrm2, 0.0)) / den
            rc = jnp.clip(SQC * rr, -1.0 + 1e-5, 1.0 - 1e-5)
            z = (-INV_SQRT_HS / SQC) * _ln16((1.0 + rc) / (1.0 - rc))
            zm = jnp.max(z)
            ez = jnp.exp(z - zm)
            attw = ez / jnp.sum(ez)

            w = attw * g_s
            wxn_c = jnp.maximum(w * xn_s, EPS)
            alpha = _tanh16(wxn_c / xn_s * t_s) * (w * xn_s) / wxn_c
            wxn2 = jnp.maximum(alpha * rho_s * m_s, EPS)
            gam = haw1 * _tanh16(wxn2 / m_s * A_s) * (m_s * alpha) / wxn2
            x2e = alpha * alpha * u2_s
            y2e = gam * gam * v2_s
            xye = alpha * gam * v2_s
            P2 = 1.0 + 2.0 * CURV * xye + CURV * y2e
            Q2 = 1.0 - CURV * x2e
            den2 = jnp.maximum(1.0 + 2.0 * CURV * xye + CURV * CURV * x2e * y2e, EPS)
            Ae = P2 * alpha / den2
            Be = Q2 * gam / den2
            hh2 = Ae * Ae * u2_s + (Be * Be + 2.0 * Ae * Be) * v2_s
            lam = 2.0 / jnp.maximum(1.0 - CURV * hh2, EPS)
            cA = lam * Ae
            cApB = cA + lam * Be
            denvec = jnp.where(
                lane == 0, jnp.sum(lam - 1.0),
                jnp.where(lane == 1, jnp.sum(q_s), 0.0))
            out_v[k, pl.ds(2 * HS, LANES)] = denvec

            hacc = [jnp.zeros((LANES,), jnp.float32) for _ in range(8)]
            cacc = [jnp.zeros((LANES,), jnp.float32) for _ in range(8)]
            for e in range(DEG):
                r = k * DEG + e
                cAe = cA[e]
                cApBe = cApB[e]
                for c in range(8):
                    uu = rows_v[r, pl.ds(HS + c * LANES, LANES)]
                    coef = jnp.where(uu > 0.0, cApBe, cAe)
                    hacc[c] = hacc[c] + coef * uu
                    cacc[c] = cacc[c] + rows_v[r, pl.ds(2 * HS + c * LANES, LANES)]
            for c in range(8):
                out_v[k, pl.ds(c * LANES, LANES)] = hacc[c]
                out_v[k, pl.ds(HS + c * LANES, LANES)] = cacc[c]
            return ()

        lax.fori_loop(0, GD, dst_body, (), unroll=False)
        pltpu.sync_copy(out_v, O_hbm.at[pl.ds(d0, GD)])
        return ()

    lax.fori_loop(0, NGROUP, group, (), unroll=False)


def _stage2(T, Xt_pad, src_pad):
    mesh = plsc.VectorSubcoreMesh(core_axis_name="c", subcore_axis_name="s",
                                  num_cores=NC, num_subcores=NS)
    kfn = pl.kernel(
        _sc_edge_body,
        out_type=jax.ShapeDtypeStruct((NPAD, OW), jnp.float32),
        mesh=mesh,
        scratch_types=[
            pltpu.VMEM((GD * DEG,), jnp.int32),
            pltpu.VMEM((GD * DEG, TW), jnp.float32),
            pltpu.VMEM((GD, XW), jnp.float32),
            pltpu.VMEM((GD, OW), jnp.float32),
            pltpu.SemaphoreType.DMA,
        ],
        compiler_params=pltpu.CompilerParams(needs_layout_passes=False),
        interpret=_INTERPRET,
    )
    return kfn(T, Xt_pad, src_pad)


def _stage3_body(O_ref, WXI_ref, U_iou_ref, b_iou_ref, h_ref, c_ref):
    O = O_ref[...]
    h_num = O[:, 0:HS]
    c_num = O[:, HS:2 * HS]
    h_den = O[:, 2 * HS:2 * HS + 1]
    c_den = O[:, 2 * HS + 1:2 * HS + 2]

    def midfin(num, den):
        mid = num / jnp.clip(den, EPS)
        n = _norm(mid)
        return jnp.tanh(0.5 * _artanh(SQC * n)) * mid / (SQC * n)

    h_tild = midfin(h_num, h_den)
    c_red = midfin(c_num, c_den)

    iou1 = _mobius_add(WXI_ref[...], _mobius_matvec(U_iou_ref[...], h_tild))
    iou = _mobius_add(iou1, b_iou_ref[...])
    i = iou[:, 0:HS]
    o = iou[:, HS:2 * HS]
    uu = iou[:, 2 * HS:3 * HS]
    i = _sigmoid(_logmap0(i))
    o = _sigmoid(_logmap0(o))
    uu = jnp.tanh(_logmap0(uu))
    c_new = _mobius_add(_mobius_pw(i, uu), c_red)
    h_new = _mobius_pw(o, jnp.tanh(_logmap0(c_new)))
    h_ref[...] = h_new
    c_ref[...] = c_new


def _stage3(O, WXI, U_iou, b_iou):
    BN = 1000
    grid = (N // BN,)
    row = lambda i: (i, 0)
    whole = lambda i: (0, 0)
    return pl.pallas_call(
        _stage3_body,
        grid=grid,
        in_specs=[
            pl.BlockSpec((BN, OW), row),
            pl.BlockSpec((BN, 3 * HS), row),
            pl.BlockSpec((3 * HS, HS), whole),
            pl.BlockSpec((1, 3 * HS), whole),
        ],
        out_specs=[
            pl.BlockSpec((BN, HS), row),
            pl.BlockSpec((BN, HS), row),
        ],
        out_shape=[
            jax.ShapeDtypeStruct((N, HS), jnp.float32),
            jax.ShapeDtypeStruct((N, HS), jnp.float32),
        ],
        interpret=_INTERPRET,
    )(O, WXI, U_iou, b_iou)


def kernel(x, h1, c1, del_t, edge_index, W_iou, U_iou, b_iou, U_f,
           W_q, b_q, W_k, b_k, W_c, b_c, a, b, haw_1, haw_2):
    dt = del_t.reshape(N, 1)
    sc = lambda s: s.reshape(1, 1).astype(jnp.float32)
    T, Xt, WXI = _stage1(x, h1, c1, dt, W_q, b_q, W_k, b_k, W_c, b_c,
                         U_f, W_iou, sc(a), sc(b), sc(haw_1), sc(haw_2))

    src = edge_index[0]
    src_pad = jnp.concatenate(
        [src, jnp.zeros((NPAD * DEG - N * DEG,), jnp.int32)])
    Xt_pad = jnp.concatenate(
        [Xt, jnp.zeros((NPAD - N, XW), jnp.float32)], axis=0)
    O = _stage2(T, Xt_pad, src_pad)

    h_new, c_new = _stage3(O[:N], WXI, U_iou, b_iou)
    return h_new, c_new


# speedup vs baseline: 8.5475x; 1.2900x over previous
import functools

import jax
import jax.numpy as jnp
import numpy as np
from jax import lax
from jax.experimental import pallas as pl
from jax.experimental.pallas import tpu as pltpu
from jax.experimental.pallas import tpu_sc as plsc

N = 10000
DEG = 16
XS = 128
HS = 128
CURV = 1.0
SQC = float(np.sqrt(CURV))
EPS = 1e-15
INV_SQRT_HS = float(1.0 / np.sqrt(HS))

NC = 2
NS = 16
NW = NC * NS
LANES = 16

PERW = 320
NPAD = NW * PERW
GD = 8
NGROUP = PERW // GD

TW = 512
XW = 256
OW = 384

S_XN, S_T, S_G, S_M, S_A, S_Q, S_KH2, S_U2, S_V2, S_RHO, S_HAW1 = range(11)

_INTERPRET = False
_PREC = lax.Precision.HIGHEST


def _artanh(x):
    return 0.5 * jnp.log((1.0 + jnp.clip(x, -1.0 + 1e-5, 1.0 - 1e-5)) /
                         (1.0 - jnp.clip(x, -1.0 + 1e-5, 1.0 - 1e-5)))


def _norm(x):
    return jnp.clip(jnp.sqrt(jnp.sum(x * x, -1, keepdims=True)), EPS)


def _mobius_add(x, y):
    x2 = jnp.sum(x * x, -1, keepdims=True)
    y2 = jnp.sum(y * y, -1, keepdims=True)
    xy = jnp.sum(x * y, -1, keepdims=True)
    num = (1.0 + 2.0 * CURV * xy + CURV * y2) * x + (1.0 - CURV * x2) * y
    den = 1.0 + 2.0 * CURV * xy + CURV * CURV * x2 * y2
    return num / jnp.clip(den, EPS)


def _expmap0(u):
    n = _norm(u)
    return jnp.tanh(SQC * n) * u / (SQC * n)


def _logmap0(x):
    n = _norm(x)
    return _artanh(SQC * n) * x / (SQC * n)


def _mobius_matvec(M, x):
    mx = lax.dot_general(x, M, (((1,), (1,)), ((), ())), precision=_PREC)
    xn = _norm(x)
    mxn = _norm(mx)
    return jnp.tanh(mxn / xn * _artanh(SQC * xn)) * mx / (mxn * SQC)


def _mobius_pw(w, x):
    wx = w * x
    xn = _norm(x)
    wxn = _norm(wx)
    return jnp.tanh(wxn / xn * _artanh(SQC * xn)) * wx / (wxn * SQC)


def _sigmoid(x):
    return 1.0 / (1.0 + jnp.exp(-x))


def _stage1_body(x_ref, h1_ref, c1_ref, dt_ref,
                 W_q_ref, b_q_ref, W_k_ref, b_k_ref, W_c_ref, b_c_ref,
                 U_f_ref, W_iou_ref, a_ref, b_ref, h1s_ref, h2s_ref,
                 T_ref, Xt_ref, WXI_ref):
    x = x_ref[...]
    h1 = h1_ref[...]
    c1 = c1_ref[...]
    dt = dt_ref[...]
    a = a_ref[0, 0]
    b = b_ref[0, 0]
    haw_1 = h1s_ref[0, 0]
    haw_2 = h2s_ref[0, 0]

    ex = _expmap0(x)
    x_ = _mobius_add(_mobius_matvec(W_q_ref[...], ex), b_q_ref[...])
    x2 = jnp.sum(x_ * x_, -1, keepdims=True)
    kh = _mobius_add(_mobius_matvec(W_k_ref[...], h1), b_k_ref[...])
    kh2 = jnp.sum(kh * kh, -1, keepdims=True)

    xn = jnp.clip(jnp.sqrt(jnp.sum(h1 * h1, -1, keepdims=True)), EPS)
    t = _artanh(SQC * xn)
    u = h1 / (xn * SQC)
    v = jnp.maximum(u, 0.0)
    u2 = jnp.sum(u * u, -1, keepdims=True)
    v2 = jnp.sum(v * v, -1, keepdims=True)
    rho = jnp.sqrt(v2)
    m = jnp.exp(-haw_2 * dt / 60.0)
    A = _artanh(SQC * m)
    g = b * jnp.exp(-a * dt)

    cc = _mobius_add(_mobius_matvec(W_c_ref[...], c1), b_c_ref[...])
    c_sk = _expmap0(jnp.tanh(_logmap0(cc)))
    c_sk_hat = _mobius_pw(c_sk, g)
    c_Tk = _mobius_add(-c_sk, c1)
    c_kt = _mobius_add(c_Tk, c_sk_hat)
    f = _sigmoid(_logmap0(_mobius_matvec(U_f_ref[...], h1)))
    v_c = _mobius_pw(f, c_kt)
    lam_c = 2.0 / jnp.clip(1.0 - CURV * jnp.sum(v_c * v_c, -1, keepdims=True), EPS)
    p = lam_c * v_c
    q = lam_c - 1.0

    bn = x.shape[0]
    haw1_col = jnp.full((bn, 1), haw_1, jnp.float32)
    zpad = jnp.zeros((bn, TW - 3 * HS - 11), jnp.float32)
    scal = jnp.concatenate(
        [xn, t, g, m, A, q, kh2, u2, v2, rho, haw1_col, zpad], axis=1)
    T_ref[...] = jnp.concatenate([kh, u, p, scal], axis=1)
    Xt_ref[...] = jnp.concatenate(
        [x_, x2, jnp.zeros((bn, XW - HS - 1), jnp.float32)], axis=1)
    WXI_ref[...] = _mobius_matvec(W_iou_ref[...], ex)


def _stage1(x, h1, c1, dt, W_q, b_q, W_k, b_k, W_c, b_c, U_f, W_iou,
            a, b, haw_1, haw_2):
    BN = 1000
    grid = (N // BN,)
    row = lambda i: (i, 0)
    whole = lambda i: (0, 0)
    return pl.pallas_call(
        _stage1_body,
        grid=grid,
        in_specs=[
            pl.BlockSpec((BN, XS), row),
            pl.BlockSpec((BN, HS), row),
            pl.BlockSpec((BN, HS), row),
            pl.BlockSpec((BN, 1), row),
            pl.BlockSpec((HS, XS), whole),
            pl.BlockSpec((1, HS), whole),
            pl.BlockSpec((HS, HS), whole),
            pl.BlockSpec((1, HS), whole),
            pl.BlockSpec((HS, HS), whole),
            pl.BlockSpec((1, HS), whole),
            pl.BlockSpec((HS, HS), whole),
            pl.BlockSpec((3 * HS, XS), whole),
            pl.BlockSpec((1, 1), whole),
            pl.BlockSpec((1, 1), whole),
            pl.BlockSpec((1, 1), whole),
            pl.BlockSpec((1, 1), whole),
        ],
        out_specs=[
            pl.BlockSpec((BN, TW), row),
            pl.BlockSpec((BN, XW), row),
            pl.BlockSpec((BN, 3 * HS), row),
        ],
        out_shape=[
            jax.ShapeDtypeStruct((N, TW), jnp.float32),
            jax.ShapeDtypeStruct((N, XW), jnp.float32),
            jax.ShapeDtypeStruct((N, 3 * HS), jnp.float32),
        ],
        interpret=_INTERPRET,
    )(x, h1, c1, dt, W_q, b_q, W_k, b_k, W_c, b_c, U_f, W_iou,
      a, b, haw_1, haw_2)


def _ln16(x):
    bits = plsc.bitcast(x, jnp.int32)
    e = ((bits >> 23) & 0xFF) - 127
    m = plsc.bitcast((bits & 0x007FFFFF) | 0x3F800000, jnp.float32)
    big = m > 1.4142135623730951
    m = jnp.where(big, m * 0.5, m)
    e = e + big.astype(jnp.int32)
    s = (m - 1.0) / (m + 1.0)
    s2 = s * s
    poly = 1.0 + s2 * (0.3333333333 + s2 * (0.2 + s2 * (0.14285714 + s2 * 0.11111111)))
    return 2.0 * s * poly + e.astype(jnp.float32) * 0.6931471805599453


def _sqrt16(x):
    xc = jnp.maximum(x, 1e-35)
    y = plsc.bitcast(jnp.int32(0x5F3759DF) - (plsc.bitcast(xc, jnp.int32) >> 1),
                     jnp.float32)
    for _ in range(3):
        y = y * (1.5 - 0.5 * xc * y * y)
    return x * y


def _tanh16(z):
    ez = jnp.exp(-2.0 * z)
    return (1.0 - ez) / (1.0 + ez)


HG = GD // 2


def _sc_edge_body(T_hbm, Xt_hbm, SRC_hbm, O_hbm,
                  idxa_v, idxb_v, rows_v, xt_v, out_v, sema, semb):
    wid = lax.axis_index("s") * NC + lax.axis_index("c")
    base = wid * PERW
    lane = lax.iota(jnp.int32, LANES)
    HROWS = HG * DEG

    def start_half(gi, half, idx_v, sem):
        d0 = base + gi * GD + half * HG
        pltpu.sync_copy(SRC_hbm.at[pl.ds(d0 * DEG, HROWS)], idx_v)
        half_rows = rows_v.at[pl.ds(half * HROWS, HROWS)]
        pltpu.async_copy(T_hbm.at[idx_v], half_rows, sem)

    def wait_half(half, sem):
        half_rows = rows_v.at[pl.ds(half * HROWS, HROWS)]
        pltpu.make_async_copy(T_hbm.at[pl.ds(0, HROWS)], half_rows, sem).wait()

    start_half(0, 0, idxa_v, sema)
    start_half(0, 1, idxb_v, semb)

    def group(gi, _):
        d0 = base + gi * GD
        pltpu.sync_copy(Xt_hbm.at[pl.ds(d0, GD)], xt_v)

        def dst_body(k, _):
            xch = [xt_v[k, pl.ds(c * LANES, LANES)] for c in range(8)]
            dots = jnp.zeros((LANES,), jnp.float32)
            for e in range(DEG):
                r = k * DEG + e
                acc = rows_v[r, pl.ds(0, LANES)] * xch[0]
                for c in range(1, 8):
                    acc = acc + rows_v[r, pl.ds(c * LANES, LANES)] * xch[c]
                dots = jnp.where(lane == e, jnp.sum(acc), dots)

            ridx = lane + (k * DEG)

            def gcol(slot):
                col = jnp.full((LANES,), 3 * HS + slot, jnp.int32)
                return plsc.load_gather(rows_v, [ridx, col])

            xn_s = gcol(S_XN)
            t_s = gcol(S_T)
            g_s = gcol(S_G)
            m_s = gcol(S_M)
            A_s = gcol(S_A)
            q_s = gcol(S_Q)
            kh2_s = gcol(S_KH2)
            u2_s = gcol(S_U2)
            v2_s = gcol(S_V2)
            rho_s = gcol(S_RHO)
            haw1 = gcol(S_HAW1)

            x2d = xt_v[k, pl.ds(HS, LANES)][0]
            P = 1.0 - 2.0 * CURV * dots + CURV * kh2_s
            Q = 1.0 - CURV * x2d
            den = jnp.maximum(1.0 - 2.0 * CURV * dots + CURV * CURV * x2d * kh2_s, EPS)
            nrm2 = P * P * x2d + Q * Q * kh2_s - 2.0 * P * Q * dots
            rr = _sqrt16(jnp.maximum(nrm2, 0.0)) / den
            rc = jnp.clip(SQC * rr, -1.0 + 1e-5, 1.0 - 1e-5)
            z = (-INV_SQRT_HS / SQC) * _ln16((1.0 + rc) / (1.0 - rc))
            zm = jnp.max(z)
            ez = jnp.exp(z - zm)
            attw = ez / jnp.sum(ez)

            w = attw * g_s
            wxn_c = jnp.maximum(w * xn_s, EPS)
            alpha = _tanh16(wxn_c / xn_s * t_s) * (w * xn_s) / wxn_c
            wxn2 = jnp.maximum(alpha * rho_s * m_s, EPS)
            gam = haw1 * _tanh16(wxn2 / m_s * A_s) * (m_s * alpha) / wxn2
            x2e = alpha * alpha * u2_s
            y2e = gam * gam * v2_s
            xye = alpha * gam * v2_s
            P2 = 1.0 + 2.0 * CURV * xye + CURV * y2e
            Q2 = 1.0 - CURV * x2e
            den2 = jnp.maximum(1.0 + 2.0 * CURV * xye + CURV * CURV * x2e * y2e, EPS)
            Ae = P2 * alpha / den2
            Be = Q2 * gam / den2
            hh2 = Ae * Ae * u2_s + (Be * Be + 2.0 * Ae * Be) * v2_s
            lam = 2.0 / jnp.maximum(1.0 - CURV * hh2, EPS)
            cA = lam * Ae
            cApB = cA + lam * Be
            denvec = jnp.where(
                lane == 0, jnp.sum(lam - 1.0),
                jnp.where(lane == 1, jnp.sum(q_s), 0.0))
            out_v[k, pl.ds(2 * HS, LANES)] = denvec

            hacc = [jnp.zeros((LANES,), jnp.float32) for _ in range(8)]
            cacc = [jnp.zeros((LANES,), jnp.float32) for _ in range(8)]
            for e in range(DEG):
                r = k * DEG + e
                cAe = cA[e]
                cApBe = cApB[e]
                for c in range(8):
                    uu = rows_v[r, pl.ds(HS + c * LANES, LANES)]
                    coef = jnp.where(uu > 0.0, cApBe, cAe)
                    hacc[c] = hacc[c] + coef * uu
                    cacc[c] = cacc[c] + rows_v[r, pl.ds(2 * HS + c * LANES, LANES)]
            for c in range(8):
                out_v[k, pl.ds(c * LANES, LANES)] = hacc[c]
                out_v[k, pl.ds(HS + c * LANES, LANES)] = cacc[c]
            return ()

        wait_half(0, sema)
        lax.fori_loop(0, HG, dst_body, (), unroll=False)

        @pl.when(gi + 1 < NGROUP)
        def _():
            start_half(gi + 1, 0, idxa_v, sema)

        wait_half(1, semb)
        lax.fori_loop(HG, GD, dst_body, (), unroll=False)

        @pl.when(gi + 1 < NGROUP)
        def _():
            start_half(gi + 1, 1, idxb_v, semb)

        pltpu.sync_copy(out_v, O_hbm.at[pl.ds(d0, GD)])
        return ()

    lax.fori_loop(0, NGROUP, group, (), unroll=False)


def _stage2(T, Xt_pad, src_pad):
    mesh = plsc.VectorSubcoreMesh(core_axis_name="c", subcore_axis_name="s",
                                  num_cores=NC, num_subcores=NS)
    kfn = pl.kernel(
        _sc_edge_body,
        out_type=jax.ShapeDtypeStruct((NPAD, OW), jnp.float32),
        mesh=mesh,
        scratch_types=[
            pltpu.VMEM((GD * DEG // 2,), jnp.int32),
            pltpu.VMEM((GD * DEG // 2,), jnp.int32),
            pltpu.VMEM((GD * DEG, TW), jnp.float32),
            pltpu.VMEM((GD, XW), jnp.float32),
            pltpu.VMEM((GD, OW), jnp.float32),
            pltpu.SemaphoreType.DMA,
            pltpu.SemaphoreType.DMA,
        ],
        compiler_params=pltpu.CompilerParams(needs_layout_passes=False),
        interpret=_INTERPRET,
    )
    return kfn(T, Xt_pad, src_pad)


def _stage3_body(O_ref, WXI_ref, U_iou_ref, b_iou_ref, h_ref, c_ref):
    O = O_ref[...]
    h_num = O[:, 0:HS]
    c_num = O[:, HS:2 * HS]
    h_den = O[:, 2 * HS:2 * HS + 1]
    c_den = O[:, 2 * HS + 1:2 * HS + 2]

    def midfin(num, den):
        mid = num / jnp.clip(den, EPS)
        n = _norm(mid)
        return jnp.tanh(0.5 * _artanh(SQC * n)) * mid / (SQC * n)

    h_tild = midfin(h_num, h_den)
    c_red = midfin(c_num, c_den)

    iou1 = _mobius_add(WXI_ref[...], _mobius_matvec(U_iou_ref[...], h_tild))
    iou = _mobius_add(iou1, b_iou_ref[...])
    i = iou[:, 0:HS]
    o = iou[:, HS:2 * HS]
    uu = iou[:, 2 * HS:3 * HS]
    i = _sigmoid(_logmap0(i))
    o = _sigmoid(_logmap0(o))
    uu = jnp.tanh(_logmap0(uu))
    c_new = _mobius_add(_mobius_pw(i, uu), c_red)
    h_new = _mobius_pw(o, jnp.tanh(_logmap0(c_new)))
    h_ref[...] = h_new
    c_ref[...] = c_new


def _stage3(O, WXI, U_iou, b_iou):
    BN = 1000
    grid = (N // BN,)
    row = lambda i: (i, 0)
    whole = lambda i: (0, 0)
    return pl.pallas_call(
        _stage3_body,
        grid=grid,
        in_specs=[
            pl.BlockSpec((BN, OW), row),
            pl.BlockSpec((BN, 3 * HS), row),
            pl.BlockSpec((3 * HS, HS), whole),
            pl.BlockSpec((1, 3 * HS), whole),
        ],
        out_specs=[
            pl.BlockSpec((BN, HS), row),
            pl.BlockSpec((BN, HS), row),
        ],
        out_shape=[
            jax.ShapeDtypeStruct((N, HS), jnp.float32),
            jax.ShapeDtypeStruct((N, HS), jnp.float32),
        ],
        interpret=_INTERPRET,
    )(O, WXI, U_iou, b_iou)


def kernel(x, h1, c1, del_t, edge_index, W_iou, U_iou, b_iou, U_f,
           W_q, b_q, W_k, b_k, W_c, b_c, a, b, haw_1, haw_2):
    dt = del_t.reshape(N, 1)
    sc = lambda s: s.reshape(1, 1).astype(jnp.float32)
    T, Xt, WXI = _stage1(x, h1, c1, dt, W_q, b_q, W_k, b_k, W_c, b_c,
                         U_f, W_iou, sc(a), sc(b), sc(haw_1), sc(haw_2))

    src = edge_index[0]
    src_pad = jnp.concatenate(
        [src, jnp.zeros((NPAD * DEG - N * DEG,), jnp.int32)])
    Xt_pad = jnp.concatenate(
        [Xt, jnp.zeros((NPAD - N, XW), jnp.float32)], axis=0)
    O = _stage2(T, Xt_pad, src_pad)

    h_new, c_new = _stage3(O[:N], WXI, U_iou, b_iou)
    return h_new, c_new


# speedup vs baseline: 9.0555x; 1.0594x over previous
import functools

import jax
import jax.numpy as jnp
import numpy as np
from jax import lax
from jax.experimental import pallas as pl
from jax.experimental.pallas import tpu as pltpu
from jax.experimental.pallas import tpu_sc as plsc

N = 10000
DEG = 16
XS = 128
HS = 128
CURV = 1.0
SQC = float(np.sqrt(CURV))
EPS = 1e-15
INV_SQRT_HS = float(1.0 / np.sqrt(HS))

NC = 2
NS = 16
NW = NC * NS
LANES = 16

PERW = 320
NPAD = NW * PERW
GD = 8
NGROUP = PERW // GD

TW = 384
XW = 256
OW = 384

S_XN, S_T, S_G, S_M, S_A, S_Q, S_KH2, S_U2, S_V2, S_RHO, S_HAW1 = range(11)

_INTERPRET = False
_PREC = lax.Precision.HIGHEST


def _artanh(x):
    return 0.5 * jnp.log((1.0 + jnp.clip(x, -1.0 + 1e-5, 1.0 - 1e-5)) /
                         (1.0 - jnp.clip(x, -1.0 + 1e-5, 1.0 - 1e-5)))


def _norm(x):
    return jnp.clip(jnp.sqrt(jnp.sum(x * x, -1, keepdims=True)), EPS)


def _mobius_add(x, y):
    x2 = jnp.sum(x * x, -1, keepdims=True)
    y2 = jnp.sum(y * y, -1, keepdims=True)
    xy = jnp.sum(x * y, -1, keepdims=True)
    num = (1.0 + 2.0 * CURV * xy + CURV * y2) * x + (1.0 - CURV * x2) * y
    den = 1.0 + 2.0 * CURV * xy + CURV * CURV * x2 * y2
    return num / jnp.clip(den, EPS)


def _expmap0(u):
    n = _norm(u)
    return jnp.tanh(SQC * n) * u / (SQC * n)


def _logmap0(x):
    n = _norm(x)
    return _artanh(SQC * n) * x / (SQC * n)


def _mobius_matvec(M, x):
    mx = lax.dot_general(x, M, (((1,), (1,)), ((), ())), precision=_PREC)
    xn = _norm(x)
    mxn = _norm(mx)
    return jnp.tanh(mxn / xn * _artanh(SQC * xn)) * mx / (mxn * SQC)


def _mobius_pw(w, x):
    wx = w * x
    xn = _norm(x)
    wxn = _norm(wx)
    return jnp.tanh(wxn / xn * _artanh(SQC * xn)) * wx / (wxn * SQC)


def _sigmoid(x):
    return 1.0 / (1.0 + jnp.exp(-x))


def _stage1_body(x_ref, h1_ref, c1_ref, dt_ref,
                 W_q_ref, b_q_ref, W_k_ref, b_k_ref, W_c_ref, b_c_ref,
                 U_f_ref, W_iou_ref, a_ref, b_ref, h1s_ref, h2s_ref,
                 T_ref, Xt_ref, WXI_ref):
    x = x_ref[...]
    h1 = h1_ref[...]
    c1 = c1_ref[...]
    dt = dt_ref[...]
    a = a_ref[0, 0]
    b = b_ref[0, 0]
    haw_1 = h1s_ref[0, 0]
    haw_2 = h2s_ref[0, 0]

    ex = _expmap0(x)
    x_ = _mobius_add(_mobius_matvec(W_q_ref[...], ex), b_q_ref[...])
    x2 = jnp.sum(x_ * x_, -1, keepdims=True)
    kh = _mobius_add(_mobius_matvec(W_k_ref[...], h1), b_k_ref[...])
    kh2 = jnp.sum(kh * kh, -1, keepdims=True)

    xn = jnp.clip(jnp.sqrt(jnp.sum(h1 * h1, -1, keepdims=True)), EPS)
    t = _artanh(SQC * xn)
    u = h1 / (xn * SQC)
    v = jnp.maximum(u, 0.0)
    u2 = jnp.sum(u * u, -1, keepdims=True)
    v2 = jnp.sum(v * v, -1, keepdims=True)
    rho = jnp.sqrt(v2)
    m = jnp.exp(-haw_2 * dt / 60.0)
    A = _artanh(SQC * m)
    g = b * jnp.exp(-a * dt)

    cc = _mobius_add(_mobius_matvec(W_c_ref[...], c1), b_c_ref[...])
    c_sk = _expmap0(jnp.tanh(_logmap0(cc)))
    c_sk_hat = _mobius_pw(c_sk, g)
    c_Tk = _mobius_add(-c_sk, c1)
    c_kt = _mobius_add(c_Tk, c_sk_hat)
    f = _sigmoid(_logmap0(_mobius_matvec(U_f_ref[...], h1)))
    v_c = _mobius_pw(f, c_kt)
    lam_c = 2.0 / jnp.clip(1.0 - CURV * jnp.sum(v_c * v_c, -1, keepdims=True), EPS)
    p = lam_c * v_c
    q = lam_c - 1.0

    bn = x.shape[0]
    haw1_col = jnp.full((bn, 1), haw_1, jnp.float32)
    zpad = jnp.zeros((bn, TW - 2 * HS - 11), jnp.float32)
    scal = jnp.concatenate(
        [xn, t, g, m, A, q, kh2, u2, v2, rho, haw1_col, zpad], axis=1)
    kb = lax.bitcast_convert_type(kh, jnp.uint32) + jnp.uint32(0x8000)
    ub = lax.bitcast_convert_type(u, jnp.uint32) + jnp.uint32(0x8000)
    packed = (kb & jnp.uint32(0xFFFF0000)) | (ub >> 16)
    packedf = lax.bitcast_convert_type(packed, jnp.float32)
    T_ref[...] = jnp.concatenate([packedf, p, scal], axis=1)
    Xt_ref[...] = jnp.concatenate(
        [x_, x2, jnp.zeros((bn, XW - HS - 1), jnp.float32)], axis=1)
    WXI_ref[...] = _mobius_matvec(W_iou_ref[...], ex)


def _stage1(x, h1, c1, dt, W_q, b_q, W_k, b_k, W_c, b_c, U_f, W_iou,
            a, b, haw_1, haw_2):
    BN = 1000
    grid = (N // BN,)
    row = lambda i: (i, 0)
    whole = lambda i: (0, 0)
    return pl.pallas_call(
        _stage1_body,
        grid=grid,
        in_specs=[
            pl.BlockSpec((BN, XS), row),
            pl.BlockSpec((BN, HS), row),
            pl.BlockSpec((BN, HS), row),
            pl.BlockSpec((BN, 1), row),
            pl.BlockSpec((HS, XS), whole),
            pl.BlockSpec((1, HS), whole),
            pl.BlockSpec((HS, HS), whole),
            pl.BlockSpec((1, HS), whole),
            pl.BlockSpec((HS, HS), whole),
            pl.BlockSpec((1, HS), whole),
            pl.BlockSpec((HS, HS), whole),
            pl.BlockSpec((3 * HS, XS), whole),
            pl.BlockSpec((1, 1), whole),
            pl.BlockSpec((1, 1), whole),
            pl.BlockSpec((1, 1), whole),
            pl.BlockSpec((1, 1), whole),
        ],
        out_specs=[
            pl.BlockSpec((BN, TW), row),
            pl.BlockSpec((BN, XW), row),
            pl.BlockSpec((BN, 3 * HS), row),
        ],
        out_shape=[
            jax.ShapeDtypeStruct((N, TW), jnp.float32),
            jax.ShapeDtypeStruct((N, XW), jnp.float32),
            jax.ShapeDtypeStruct((N, 3 * HS), jnp.float32),
        ],
        interpret=_INTERPRET,
    )(x, h1, c1, dt, W_q, b_q, W_k, b_k, W_c, b_c, U_f, W_iou,
      a, b, haw_1, haw_2)


def _ln16(x):
    bits = plsc.bitcast(x, jnp.int32)
    e = ((bits >> 23) & 0xFF) - 127
    m = plsc.bitcast((bits & 0x007FFFFF) | 0x3F800000, jnp.float32)
    big = m > 1.4142135623730951
    m = jnp.where(big, m * 0.5, m)
    e = e + big.astype(jnp.int32)
    s = (m - 1.0) / (m + 1.0)
    s2 = s * s
    poly = 1.0 + s2 * (0.3333333333 + s2 * (0.2 + s2 * (0.14285714 + s2 * 0.11111111)))
    return 2.0 * s * poly + e.astype(jnp.float32) * 0.6931471805599453


def _sqrt16(x):
    xc = jnp.maximum(x, 1e-35)
    y = plsc.bitcast(jnp.int32(0x5F3759DF) - (plsc.bitcast(xc, jnp.int32) >> 1),
                     jnp.float32)
    for _ in range(3):
        y = y * (1.5 - 0.5 * xc * y * y)
    return x * y


def _tanh16(z):
    ez = jnp.exp(-2.0 * z)
    return (1.0 - ez) / (1.0 + ez)


HG = GD // 2


def _sc_edge_body(T_hbm, Xt_hbm, SRC_hbm, O_hbm,
                  idxa_v, idxb_v, rows_v, xt_v, out_v, sema, semb):
    wid = lax.axis_index("s") * NC + lax.axis_index("c")
    base = wid * PERW
    lane = lax.iota(jnp.int32, LANES)
    HROWS = HG * DEG

    def start_half(gi, half, idx_v, sem):
        d0 = base + gi * GD + half * HG
        pltpu.sync_copy(SRC_hbm.at[pl.ds(d0 * DEG, HROWS)], idx_v)
        half_rows = rows_v.at[pl.ds(half * HROWS, HROWS)]
        pltpu.async_copy(T_hbm.at[idx_v], half_rows, sem)

    def wait_half(half, sem):
        half_rows = rows_v.at[pl.ds(half * HROWS, HROWS)]
        pltpu.make_async_copy(T_hbm.at[pl.ds(0, HROWS)], half_rows, sem).wait()

    start_half(0, 0, idxa_v, sema)
    start_half(0, 1, idxb_v, semb)

    def group(gi, _):
        d0 = base + gi * GD
        pltpu.sync_copy(Xt_hbm.at[pl.ds(d0, GD)], xt_v)

        def dst_body(k, _):
            xch = [xt_v[k, pl.ds(c * LANES, LANES)] for c in range(8)]
            dots = jnp.zeros((LANES,), jnp.float32)
            for e in range(DEG):
                r = k * DEG + e
                prod = []
                for c in range(8):
                    pb = plsc.bitcast(rows_v[r, pl.ds(c * LANES, LANES)],
                                      jnp.uint32)
                    khc = plsc.bitcast(pb & jnp.uint32(0xFFFF0000),
                                       jnp.float32)
                    prod.append(khc * xch[c])
                s01 = (prod[0] + prod[1]) + (prod[2] + prod[3])
                s23 = (prod[4] + prod[5]) + (prod[6] + prod[7])
                dots = jnp.where(lane == e, jnp.sum(s01 + s23), dots)

            ridx = lane + (k * DEG)

            def gcol(slot):
                col = jnp.full((LANES,), 2 * HS + slot, jnp.int32)
                return plsc.load_gather(rows_v, [ridx, col])

            xn_s = gcol(S_XN)
            t_s = gcol(S_T)
            g_s = gcol(S_G)
            m_s = gcol(S_M)
            A_s = gcol(S_A)
            q_s = gcol(S_Q)
            kh2_s = gcol(S_KH2)
            u2_s = gcol(S_U2)
            v2_s = gcol(S_V2)
            rho_s = gcol(S_RHO)
            haw1 = gcol(S_HAW1)

            x2d = xt_v[k, pl.ds(HS, LANES)][0]
            P = 1.0 - 2.0 * CURV * dots + CURV * kh2_s
            Q = 1.0 - CURV * x2d
            den = jnp.maximum(1.0 - 2.0 * CURV * dots + CURV * CURV * x2d * kh2_s, EPS)
            nrm2 = P * P * x2d + Q * Q * kh2_s - 2.0 * P * Q * dots
            rr = _sqrt16(jnp.maximum(nrm2, 0.0)) / den
            rc = jnp.clip(SQC * rr, -1.0 + 1e-5, 1.0 - 1e-5)
            z = (-INV_SQRT_HS / SQC) * _ln16((1.0 + rc) / (1.0 - rc))
            zm = jnp.max(z)
            ez = jnp.exp(z - zm)
            attw = ez / jnp.sum(ez)

            w = attw * g_s
            wxn_c = jnp.maximum(w * xn_s, EPS)
            alpha = _tanh16(wxn_c / xn_s * t_s) * (w * xn_s) / wxn_c
            wxn2 = jnp.maximum(alpha * rho_s * m_s, EPS)
            gam = haw1 * _tanh16(wxn2 / m_s * A_s) * (m_s * alpha) / wxn2
            x2e = alpha * alpha * u2_s
            y2e = gam * gam * v2_s
            xye = alpha * gam * v2_s
            P2 = 1.0 + 2.0 * CURV * xye + CURV * y2e
            Q2 = 1.0 - CURV * x2e
            den2 = jnp.maximum(1.0 + 2.0 * CURV * xye + CURV * CURV * x2e * y2e, EPS)
            Ae = P2 * alpha / den2
            Be = Q2 * gam / den2
            hh2 = Ae * Ae * u2_s + (Be * Be + 2.0 * Ae * Be) * v2_s
            lam = 2.0 / jnp.maximum(1.0 - CURV * hh2, EPS)
            cA = lam * Ae
            cApB = cA + lam * Be
            denvec = jnp.where(
                lane == 0, jnp.sum(lam - 1.0),
                jnp.where(lane == 1, jnp.sum(q_s), 0.0))
            out_v[k, pl.ds(2 * HS, LANES)] = denvec

            hacc = [jnp.zeros((LANES,), jnp.float32) for _ in range(8)]
            cacc = [jnp.zeros((LANES,), jnp.float32) for _ in range(8)]
            for e in range(DEG):
                r = k * DEG + e
                cAe = cA[e]
                cApBe = cApB[e]
                for c in range(8):
                    pb = plsc.bitcast(rows_v[r, pl.ds(c * LANES, LANES)],
                                      jnp.uint32)
                    uu = plsc.bitcast(pb << 16, jnp.float32)
                    coef = jnp.where(uu > 0.0, cApBe, cAe)
                    hacc[c] = hacc[c] + coef * uu
                    cacc[c] = cacc[c] + rows_v[r, pl.ds(HS + c * LANES, LANES)]
            for c in range(8):
                out_v[k, pl.ds(c * LANES, LANES)] = hacc[c]
                out_v[k, pl.ds(HS + c * LANES, LANES)] = cacc[c]
            return ()

        wait_half(0, sema)
        lax.fori_loop(0, HG, dst_body, (), unroll=False)

        @pl.when(gi + 1 < NGROUP)
        def _():
            start_half(gi + 1, 0, idxa_v, sema)

        wait_half(1, semb)
        lax.fori_loop(HG, GD, dst_body, (), unroll=False)

        @pl.when(gi + 1 < NGROUP)
        def _():
            start_half(gi + 1, 1, idxb_v, semb)

        pltpu.sync_copy(out_v, O_hbm.at[pl.ds(d0, GD)])
        return ()

    lax.fori_loop(0, NGROUP, group, (), unroll=False)


def _stage2(T, Xt_pad, src_pad):
    mesh = plsc.VectorSubcoreMesh(core_axis_name="c", subcore_axis_name="s",
                                  num_cores=NC, num_subcores=NS)
    kfn = pl.kernel(
        _sc_edge_body,
        out_type=jax.ShapeDtypeStruct((NPAD, OW), jnp.float32),
        mesh=mesh,
        scratch_types=[
            pltpu.VMEM((GD * DEG // 2,), jnp.int32),
            pltpu.VMEM((GD * DEG // 2,), jnp.int32),
            pltpu.VMEM((GD * DEG, TW), jnp.float32),
            pltpu.VMEM((GD, XW), jnp.float32),
            pltpu.VMEM((GD, OW), jnp.float32),
            pltpu.SemaphoreType.DMA,
            pltpu.SemaphoreType.DMA,
        ],
        compiler_params=pltpu.CompilerParams(needs_layout_passes=False),
        interpret=_INTERPRET,
    )
    return kfn(T, Xt_pad, src_pad)


def _stage3_body(O_ref, WXI_ref, U_iou_ref, b_iou_ref, h_ref, c_ref):
    O = O_ref[...]
    h_num = O[:, 0:HS]
    c_num = O[:, HS:2 * HS]
    h_den = O[:, 2 * HS:2 * HS + 1]
    c_den = O[:, 2 * HS + 1:2 * HS + 2]

    def midfin(num, den):
        mid = num / jnp.clip(den, EPS)
        n = _norm(mid)
        return jnp.tanh(0.5 * _artanh(SQC * n)) * mid / (SQC * n)

    h_tild = midfin(h_num, h_den)
    c_red = midfin(c_num, c_den)

    iou1 = _mobius_add(WXI_ref[...], _mobius_matvec(U_iou_ref[...], h_tild))
    iou = _mobius_add(iou1, b_iou_ref[...])
    i = iou[:, 0:HS]
    o = iou[:, HS:2 * HS]
    uu = iou[:, 2 * HS:3 * HS]
    i = _sigmoid(_logmap0(i))
    o = _sigmoid(_logmap0(o))
    uu = jnp.tanh(_logmap0(uu))
    c_new = _mobius_add(_mobius_pw(i, uu), c_red)
    h_new = _mobius_pw(o, jnp.tanh(_logmap0(c_new)))
    h_ref[...] = h_new
    c_ref[...] = c_new


def _stage3(O, WXI, U_iou, b_iou):
    BN = 1000
    grid = (N // BN,)
    row = lambda i: (i, 0)
    whole = lambda i: (0, 0)
    return pl.pallas_call(
        _stage3_body,
        grid=grid,
        in_specs=[
            pl.BlockSpec((BN, OW), row),
            pl.BlockSpec((BN, 3 * HS), row),
            pl.BlockSpec((3 * HS, HS), whole),
            pl.BlockSpec((1, 3 * HS), whole),
        ],
        out_specs=[
            pl.BlockSpec((BN, HS), row),
            pl.BlockSpec((BN, HS), row),
        ],
        out_shape=[
            jax.ShapeDtypeStruct((N, HS), jnp.float32),
            jax.ShapeDtypeStruct((N, HS), jnp.float32),
        ],
        interpret=_INTERPRET,
    )(O, WXI, U_iou, b_iou)


def kernel(x, h1, c1, del_t, edge_index, W_iou, U_iou, b_iou, U_f,
           W_q, b_q, W_k, b_k, W_c, b_c, a, b, haw_1, haw_2):
    dt = del_t.reshape(N, 1)
    sc = lambda s: s.reshape(1, 1).astype(jnp.float32)
    T, Xt, WXI = _stage1(x, h1, c1, dt, W_q, b_q, W_k, b_k, W_c, b_c,
                         U_f, W_iou, sc(a), sc(b), sc(haw_1), sc(haw_2))

    src = edge_index[0]
    src_pad = jnp.concatenate(
        [src, jnp.zeros((NPAD * DEG - N * DEG,), jnp.int32)])
    Xt_pad = jnp.concatenate(
        [Xt, jnp.zeros((NPAD - N, XW), jnp.float32)], axis=0)
    O = _stage2(T, Xt_pad, src_pad)

    h_new, c_new = _stage3(O[:N], WXI, U_iou, b_iou)
    return h_new, c_new
